# baseline (speedup 1.0000x reference)
import jax
import jax.numpy as jnp
from jax import lax
from jax.experimental import pallas as pl
from jax.experimental.pallas import tpu as pltpu
from jax.experimental.pallas import tpu_sc as plsc

B = 16384
D = 128
NC = 2
NS = 16
NW = NC * NS
R = B // NW
C = 128
NCHUNK = R // C

BITREV = (0, 8, 4, 12, 2, 10, 6, 14, 1, 9, 5, 13, 3, 11, 7, 15)


def _bpr_body(uidx_hbm, iidx_hbm, eu_hbm, ei_hbm, out_hbm,
              idx_u, idx_i, u0, u1, i0, i1, out_v,
              sem_u0, sem_u1, sem_i0, sem_i1):
    wid = lax.axis_index("s") * NC + lax.axis_index("c")

    cu = pltpu.async_copy(uidx_hbm.at[wid], idx_u, sem_u0)
    ci = pltpu.async_copy(iidx_hbm.at[wid], idx_i, sem_i0)
    cu.wait()
    ci.wait()

    lanes = lax.iota(jnp.int32, 16)
    perms = [lanes ^ s for s in (8, 4, 2, 1)]

    u_bufs, i_bufs = (u0, u1), (i0, i1)
    sems_u, sems_i = (sem_u0, sem_u1), (sem_i0, sem_i1)

    def fire(c):
        b = c % 2
        return (pltpu.async_copy(eu_hbm.at[idx_u.at[c]], u_bufs[b], sems_u[b]),
                pltpu.async_copy(ei_hbm.at[idx_i.at[c]], i_bufs[b], sems_i[b]))

    copies = [None] * NCHUNK
    copies[0] = fire(0)

    for c in range(NCHUNK):
        if c + 1 < NCHUNK:
            copies[c + 1] = fire(c + 1)
        for cp in copies[c]:
            cp.wait()
        u_rows, i_rows = u_bufs[c % 2], i_bufs[c % 2]

        def gbody(g, _):
            res = jnp.zeros((16,), jnp.float32)

            def dot(row):
                acc = u_rows[row, pl.ds(0, 16)] * i_rows[row, pl.ds(0, 16)]
                for k in range(1, D // 16):
                    acc = acc + (u_rows[row, pl.ds(k * 16, 16)]
                                 * i_rows[row, pl.ds(k * 16, 16)])
                return acc

            for p in range(8):
                a = dot(g * 16 + p)
                b = dot(g * 16 + p + 8)
                a = a + jnp.take_along_axis(a, perms[0], axis=0)
                b = b + jnp.take_along_axis(b, perms[0], axis=0)
                cc = jnp.where(lanes < 8, a, b)
                for perm in perms[1:]:
                    cc = cc + jnp.take_along_axis(cc, perm, axis=0)
                res = jnp.where((lanes == p) | (lanes == p + 8), cc, res)
            out_v[c, pl.ds(g * 16, 16)] = res
            return _

        lax.fori_loop(0, C // 16, gbody, None, unroll=2)

    pltpu.sync_copy(out_v, out_hbm.at[wid])


@jax.jit
def _bpr_sc(uidx, iidx, eu, ei):
    mesh = plsc.VectorSubcoreMesh(core_axis_name="c", subcore_axis_name="s",
                                  num_cores=NC, num_subcores=NS)
    k = pl.kernel(
        _bpr_body,
        out_type=jax.ShapeDtypeStruct((NW, NCHUNK, C), jnp.float32),
        mesh=mesh,
        scratch_types=[
            pltpu.VMEM((NCHUNK, C), jnp.int32),
            pltpu.VMEM((NCHUNK, C), jnp.int32),
            pltpu.VMEM((C, D), jnp.float32),
            pltpu.VMEM((C, D), jnp.float32),
            pltpu.VMEM((C, D), jnp.float32),
            pltpu.VMEM((C, D), jnp.float32),
            pltpu.VMEM((NCHUNK, C), jnp.float32),
            pltpu.SemaphoreType.DMA,
            pltpu.SemaphoreType.DMA,
            pltpu.SemaphoreType.DMA,
            pltpu.SemaphoreType.DMA,
        ],
    )
    return k(uidx, iidx, eu, ei)


def kernel(user_indices, item_indices, embed_user, embed_item):
    uidx = user_indices.astype(jnp.int32).reshape(NW, NCHUNK, C)
    iidx = item_indices.astype(jnp.int32).reshape(NW, NCHUNK, C)
    out = _bpr_sc(uidx, iidx, embed_user, embed_item)
    return out.reshape(B)


# speedup vs baseline: 1.1343x; 1.1343x over previous
import jax
import jax.numpy as jnp
from jax import lax
from jax.experimental import pallas as pl
from jax.experimental.pallas import tpu as pltpu
from jax.experimental.pallas import tpu_sc as plsc

B = 16384
D = 128
NC = 2
NS = 16
NW = NC * NS
R = B // NW
C = 128
NCHUNK = R // C

def _bpr_body(uidx_hbm, iidx_hbm, eu_hbm, ei_hbm, out_hbm,
              idx_u, idx_i, u0, u1, i0, i1, out_v,
              sem_u0, sem_u1, sem_i0, sem_i1):
    wid = lax.axis_index("s") * NC + lax.axis_index("c")

    cu = pltpu.async_copy(uidx_hbm.at[wid], idx_u, sem_u0)
    ci = pltpu.async_copy(iidx_hbm.at[wid], idx_i, sem_i0)
    cu.wait()
    ci.wait()

    lanes = lax.iota(jnp.int32, 16)
    perms = [lanes ^ s for s in (8, 4, 2, 1)]

    u_bufs, i_bufs = (u0, u1), (i0, i1)
    sems_u, sems_i = (sem_u0, sem_u1), (sem_i0, sem_i1)

    def fire(c):
        b = c % 2
        return (pltpu.async_copy(eu_hbm.at[idx_u.at[c]], u_bufs[b], sems_u[b]),
                pltpu.async_copy(ei_hbm.at[idx_i.at[c]], i_bufs[b], sems_i[b]))

    copies = [None] * NCHUNK
    copies[0] = fire(0)

    for c in range(NCHUNK):
        if c + 1 < NCHUNK:
            copies[c + 1] = fire(c + 1)
        for cp in copies[c]:
            cp.wait()
        u_rows, i_rows = u_bufs[c % 2], i_bufs[c % 2]

        def gbody(g, _):
            res = jnp.zeros((16,), jnp.float32)
            for r in range(16):
                row = g * 16 + r
                acc = u_rows[row, pl.ds(0, 16)] * i_rows[row, pl.ds(0, 16)]
                for k in range(1, D // 16):
                    acc = acc + (u_rows[row, pl.ds(k * 16, 16)]
                                 * i_rows[row, pl.ds(k * 16, 16)])
                for perm in perms:
                    acc = acc + jnp.take_along_axis(acc, perm, axis=0)
                res = jnp.where(lanes == r, acc, res)
            out_v[c, pl.ds(g * 16, 16)] = res
            return _

        lax.fori_loop(0, C // 16, gbody, None, unroll=2)

    pltpu.sync_copy(out_v, out_hbm.at[wid])


@jax.jit
def _bpr_sc(uidx, iidx, eu, ei):
    mesh = plsc.VectorSubcoreMesh(core_axis_name="c", subcore_axis_name="s",
                                  num_cores=NC, num_subcores=NS)
    k = pl.kernel(
        _bpr_body,
        out_type=jax.ShapeDtypeStruct((NW, NCHUNK, C), jnp.float32),
        mesh=mesh,
        scratch_types=[
            pltpu.VMEM((NCHUNK, C), jnp.int32),
            pltpu.VMEM((NCHUNK, C), jnp.int32),
            pltpu.VMEM((C, D), jnp.float32),
            pltpu.VMEM((C, D), jnp.float32),
            pltpu.VMEM((C, D), jnp.float32),
            pltpu.VMEM((C, D), jnp.float32),
            pltpu.VMEM((NCHUNK, C), jnp.float32),
            pltpu.SemaphoreType.DMA,
            pltpu.SemaphoreType.DMA,
            pltpu.SemaphoreType.DMA,
            pltpu.SemaphoreType.DMA,
        ],
    )
    return k(uidx, iidx, eu, ei)


def kernel(user_indices, item_indices, embed_user, embed_item):
    uidx = user_indices.astype(jnp.int32).reshape(NW, NCHUNK, C)
    iidx = item_indices.astype(jnp.int32).reshape(NW, NCHUNK, C)
    out = _bpr_sc(uidx, iidx, embed_user, embed_item)
    return out.reshape(B)
